# baseline (speedup 1.0000x reference)
import functools

import jax
import jax.numpy as jnp
from jax import lax
from jax.experimental import pallas as pl
from jax.experimental.pallas import tpu as pltpu
from jax.experimental.pallas import tpu_sc as plsc

B = 16384
D = 64
H1 = 128
H2 = 64
HALF = 50176
PCOL = 512

NC = 2
NS = 16
NW = NC * NS
B_PER_W = B // NW
CB = B_PER_W // 2


def _pack_body(lo, hi, eye, out):
    dn = (((0,), (0,)), ((), ()))
    f32 = jnp.float32
    tlo = lax.dot_general(lo[:], eye[:], dn, preferred_element_type=f32)
    thi = lax.dot_general(hi[:], eye[:], dn, preferred_element_type=f32)
    out[:] = jnp.concatenate([tlo, thi], axis=1)


def _tc_pack(tT):
    nblk = HALF // PCOL
    return pl.pallas_call(
        _pack_body,
        grid=(nblk,),
        in_specs=[pl.BlockSpec((D, PCOL), lambda i: (0, i)),
                  pl.BlockSpec((D, PCOL), lambda i: (0, i + HALF // PCOL)),
                  pl.BlockSpec((D, D), lambda i: (0, 0))],
        out_specs=pl.BlockSpec((PCOL, 2 * D), lambda i: (i, 0)),
        out_shape=jax.ShapeDtypeStruct((HALF, 2 * D), jnp.float32),
    )(tT, tT, jnp.eye(D, dtype=jnp.float32))


def _sc_gather(idx, t2):
    mesh = plsc.VectorSubcoreMesh(core_axis_name="c", subcore_axis_name="s")

    @functools.partial(
        pl.kernel,
        mesh=mesh,
        compiler_params=pltpu.CompilerParams(use_tc_tiling_on_sc=True),
        out_type=jax.ShapeDtypeStruct((B, 2 * D), jnp.float32),
        scratch_types=[
            pltpu.VMEM((CB,), jnp.int32),
            pltpu.VMEM((CB,), jnp.int32),
            pltpu.VMEM((CB, 2 * D), jnp.float32),
            pltpu.VMEM((CB, 2 * D), jnp.float32),
            pltpu.SemaphoreType.DMA,
            pltpu.SemaphoreType.DMA,
            pltpu.SemaphoreType.DMA,
            pltpu.SemaphoreType.DMA,
        ],
    )
    def k(idx_hbm, t2_hbm, out_hbm,
          idx0_v, idx1_v, rows0_v, rows1_v, g0, g1, w0, w1):
        wid = lax.axis_index("s") * NC + lax.axis_index("c")
        base = wid * B_PER_W

        def fold(iv):
            def body(i, _):
                x = iv[pl.ds(i * 16, 16)]
                iv[pl.ds(i * 16, 16)] = jnp.where(x < HALF, x, x - HALF)
                return 0
            lax.fori_loop(0, CB // 16, body, 0)

        pltpu.sync_copy(idx_hbm.at[pl.ds(base, CB)], idx0_v)
        fold(idx0_v)
        c0 = pltpu.async_copy(t2_hbm.at[idx0_v], rows0_v, g0)
        pltpu.sync_copy(idx_hbm.at[pl.ds(base + CB, CB)], idx1_v)
        fold(idx1_v)
        c1 = pltpu.async_copy(t2_hbm.at[idx1_v], rows1_v, g1)
        c0.wait()
        s0 = pltpu.async_copy(rows0_v, out_hbm.at[pl.ds(base, CB)], w0)
        c1.wait()
        s1 = pltpu.async_copy(rows1_v, out_hbm.at[pl.ds(base + CB, CB)], w1)
        s0.wait()
        s1.wait()

    return k(idx, t2)


def _mlp_body(urows, irows, uidc, iidc, uw1, ub1, uw2, ub2,
              iw1, ib1, iw2, ib2, out):
    ur = jnp.where(uidc[:] < HALF, urows[:, :D], urows[:, D:])
    ir = jnp.where(iidc[:] < HALF, irows[:, :D], irows[:, D:])
    u = jnp.dot(ur, uw1[:], preferred_element_type=jnp.float32) + ub1[:]
    u = jnp.maximum(u, 0.0)
    u = jnp.dot(u, uw2[:], preferred_element_type=jnp.float32) + ub2[:]
    u = jnp.maximum(u, 0.0)
    v = jnp.dot(ir, iw1[:], preferred_element_type=jnp.float32) + ib1[:]
    v = jnp.maximum(v, 0.0)
    v = jnp.dot(v, iw2[:], preferred_element_type=jnp.float32) + ib2[:]
    v = jnp.maximum(v, 0.0)
    out[:] = jnp.maximum(jnp.sum(u * v, axis=1), 0.0).reshape(out.shape)


BLK = 1024


def _tc_mlp(urows, irows, uid, iid, uW1, ub1, uW2, ub2, iW1, ib1, iW2, ib2):
    nblk = B // BLK
    row_spec = pl.BlockSpec((BLK, 2 * D), lambda i: (i, 0))
    idc_spec = pl.BlockSpec((BLK, 1), lambda i: (i, 0))
    w1_spec = pl.BlockSpec((D, H1), lambda i: (0, 0))
    b1_spec = pl.BlockSpec((1, H1), lambda i: (0, 0))
    w2_spec = pl.BlockSpec((H1, H2), lambda i: (0, 0))
    b2_spec = pl.BlockSpec((1, H2), lambda i: (0, 0))
    out = pl.pallas_call(
        _mlp_body,
        grid=(nblk,),
        in_specs=[row_spec, row_spec, idc_spec, idc_spec,
                  w1_spec, b1_spec, w2_spec, b2_spec,
                  w1_spec, b1_spec, w2_spec, b2_spec],
        out_specs=pl.BlockSpec((BLK // 128, 128), lambda i: (i, 0)),
        out_shape=jax.ShapeDtypeStruct((B // 128, 128), jnp.float32),
    )(urows, irows, uid.reshape(B, 1), iid.reshape(B, 1),
      uW1, ub1.reshape(1, H1), uW2, ub2.reshape(1, H2),
      iW1, ib1.reshape(1, H1), iW2, ib2.reshape(1, H2))
    return out.reshape(-1)


def kernel(uid, iid, user_table, uW1, ub1, uW2, ub2, item_table, iW1, ib1, iW2, ib2):
    uid = uid.astype(jnp.int32)
    iid = iid.astype(jnp.int32)
    ut2 = _tc_pack(user_table.T)
    urows = _sc_gather(uid, ut2)
    it2 = _tc_pack(item_table.T)
    irows = _sc_gather(iid, it2)
    return _tc_mlp(urows, irows, uid, iid,
                   uW1, ub1, uW2, ub2, iW1, ib1, iW2, ib2)


# speedup vs baseline: 1.3260x; 1.3260x over previous
import functools

import jax
import jax.numpy as jnp
from jax import lax
from jax.experimental import pallas as pl
from jax.experimental.pallas import tpu as pltpu
from jax.experimental.pallas import tpu_sc as plsc

B = 16384
D = 64
H1 = 128
H2 = 64
HALF = 50176
PCOL = 512

NC = 2
NS = 16
NW = NC * NS
B_PER_W = B // NW
CB = B_PER_W // 2


def _pack_body(ulo, uhi, ilo, ihi, eye, uout, iout):
    dn = (((0,), (0,)), ((), ()))
    f32 = jnp.float32
    tul = lax.dot_general(ulo[:], eye[:], dn, preferred_element_type=f32)
    tuh = lax.dot_general(uhi[:], eye[:], dn, preferred_element_type=f32)
    til = lax.dot_general(ilo[:], eye[:], dn, preferred_element_type=f32)
    tih = lax.dot_general(ihi[:], eye[:], dn, preferred_element_type=f32)
    uout[:] = jnp.concatenate([tul, tuh], axis=1)
    iout[:] = jnp.concatenate([til, tih], axis=1)


def _tc_pack(utT, itT):
    nblk = HALF // PCOL
    lo_spec = pl.BlockSpec((D, PCOL), lambda i: (0, i))
    hi_spec = pl.BlockSpec((D, PCOL), lambda i: (0, i + nblk))
    eye_spec = pl.BlockSpec((D, D), lambda i: (0, 0))
    out_spec = pl.BlockSpec((PCOL, 2 * D), lambda i: (i, 0))
    out_shape = jax.ShapeDtypeStruct((HALF, 2 * D), jnp.float32)
    return pl.pallas_call(
        _pack_body,
        grid=(nblk,),
        in_specs=[lo_spec, hi_spec, lo_spec, hi_spec, eye_spec],
        out_specs=[out_spec, out_spec],
        out_shape=[out_shape, out_shape],
    )(utT, utT, itT, itT, jnp.eye(D, dtype=jnp.float32))


def _sc_gather(uid, iid, ut2, it2):
    mesh = plsc.VectorSubcoreMesh(core_axis_name="c", subcore_axis_name="s")

    @functools.partial(
        pl.kernel,
        mesh=mesh,
        compiler_params=pltpu.CompilerParams(use_tc_tiling_on_sc=True),
        out_type=[
            jax.ShapeDtypeStruct((B, 2 * D), jnp.float32),
            jax.ShapeDtypeStruct((B, 2 * D), jnp.float32),
        ],
        scratch_types=[
            pltpu.VMEM((CB,), jnp.int32),
            pltpu.VMEM((CB,), jnp.int32),
            pltpu.VMEM((CB, 2 * D), jnp.float32),
            pltpu.VMEM((CB, 2 * D), jnp.float32),
            pltpu.SemaphoreType.DMA,
            pltpu.SemaphoreType.DMA,
            pltpu.SemaphoreType.DMA,
            pltpu.SemaphoreType.DMA,
        ],
    )
    def k(uid_hbm, iid_hbm, ut_hbm, it_hbm, uout_hbm, iout_hbm,
          uidx_v, iidx_v, urows_v, irows_v, gu, gi, wu, wi):
        wid = lax.axis_index("s") * NC + lax.axis_index("c")

        def fold(iv):
            def body(i, _):
                x = iv[pl.ds(i * 16, 16)]
                iv[pl.ds(i * 16, 16)] = jnp.where(x < HALF, x, x - HALF)
                return 0
            lax.fori_loop(0, CB // 16, body, 0)

        for c in range(2):
            base = wid * B_PER_W + c * CB
            pltpu.sync_copy(uid_hbm.at[pl.ds(base, CB)], uidx_v)
            fold(uidx_v)
            cu = pltpu.async_copy(ut_hbm.at[uidx_v], urows_v, gu)
            pltpu.sync_copy(iid_hbm.at[pl.ds(base, CB)], iidx_v)
            fold(iidx_v)
            ci = pltpu.async_copy(it_hbm.at[iidx_v], irows_v, gi)
            cu.wait()
            su = pltpu.async_copy(urows_v, uout_hbm.at[pl.ds(base, CB)], wu)
            ci.wait()
            si = pltpu.async_copy(irows_v, iout_hbm.at[pl.ds(base, CB)], wi)
            su.wait()
            si.wait()

    return k(uid, iid, ut2, it2)


def _mlp_body(urows, irows, uidc, iidc, uw1, ub1, uw2, ub2,
              iw1, ib1, iw2, ib2, out):
    ur = jnp.where(uidc[:] < HALF, urows[:, :D], urows[:, D:])
    ir = jnp.where(iidc[:] < HALF, irows[:, :D], irows[:, D:])
    u = jnp.dot(ur, uw1[:], preferred_element_type=jnp.float32) + ub1[:]
    u = jnp.maximum(u, 0.0)
    u = jnp.dot(u, uw2[:], preferred_element_type=jnp.float32) + ub2[:]
    u = jnp.maximum(u, 0.0)
    v = jnp.dot(ir, iw1[:], preferred_element_type=jnp.float32) + ib1[:]
    v = jnp.maximum(v, 0.0)
    v = jnp.dot(v, iw2[:], preferred_element_type=jnp.float32) + ib2[:]
    v = jnp.maximum(v, 0.0)
    out[:] = jnp.maximum(jnp.sum(u * v, axis=1), 0.0).reshape(out.shape)


BLK = 1024


def _tc_mlp(urows, irows, uid, iid, uW1, ub1, uW2, ub2, iW1, ib1, iW2, ib2):
    nblk = B // BLK
    row_spec = pl.BlockSpec((BLK, 2 * D), lambda i: (i, 0))
    idc_spec = pl.BlockSpec((BLK, 1), lambda i: (i, 0))
    w1_spec = pl.BlockSpec((D, H1), lambda i: (0, 0))
    b1_spec = pl.BlockSpec((1, H1), lambda i: (0, 0))
    w2_spec = pl.BlockSpec((H1, H2), lambda i: (0, 0))
    b2_spec = pl.BlockSpec((1, H2), lambda i: (0, 0))
    out = pl.pallas_call(
        _mlp_body,
        grid=(nblk,),
        in_specs=[row_spec, row_spec, idc_spec, idc_spec,
                  w1_spec, b1_spec, w2_spec, b2_spec,
                  w1_spec, b1_spec, w2_spec, b2_spec],
        out_specs=pl.BlockSpec((BLK // 128, 128), lambda i: (i, 0)),
        out_shape=jax.ShapeDtypeStruct((B // 128, 128), jnp.float32),
    )(urows, irows, uid.reshape(B, 1), iid.reshape(B, 1),
      uW1, ub1.reshape(1, H1), uW2, ub2.reshape(1, H2),
      iW1, ib1.reshape(1, H1), iW2, ib2.reshape(1, H2))
    return out.reshape(-1)


def kernel(uid, iid, user_table, uW1, ub1, uW2, ub2, item_table, iW1, ib1, iW2, ib2):
    uid = uid.astype(jnp.int32)
    iid = iid.astype(jnp.int32)
    ut2, it2 = _tc_pack(user_table.T, item_table.T)
    urows, irows = _sc_gather(uid, iid, ut2, it2)
    return _tc_mlp(urows, irows, uid, iid,
                   uW1, ub1, uW2, ub2, iW1, ib1, iW2, ib2)


# speedup vs baseline: 1.6103x; 1.2144x over previous
import functools

import jax
import jax.numpy as jnp
from jax import lax
from jax.experimental import pallas as pl
from jax.experimental.pallas import tpu as pltpu
from jax.experimental.pallas import tpu_sc as plsc

B = 16384
D = 64
H1 = 128
H2 = 64
HALF = 50176
PCOL = 1024

NC = 2
NS = 16
NW = NC * NS
B_PER_W = B // NW
CB = B_PER_W // 2


def _pack_body(ulo, uhi, ilo, ihi, eye, uout, iout):
    dn = (((0,), (0,)), ((), ()))
    f32 = jnp.float32
    tul = lax.dot_general(ulo[:], eye[:], dn, preferred_element_type=f32)
    tuh = lax.dot_general(uhi[:], eye[:], dn, preferred_element_type=f32)
    til = lax.dot_general(ilo[:], eye[:], dn, preferred_element_type=f32)
    tih = lax.dot_general(ihi[:], eye[:], dn, preferred_element_type=f32)
    uout[:] = jnp.concatenate([tul, tuh], axis=1)
    iout[:] = jnp.concatenate([til, tih], axis=1)


def _tc_pack(utT, itT):
    nblk = HALF // PCOL
    lo_spec = pl.BlockSpec((D, PCOL), lambda i: (0, i))
    hi_spec = pl.BlockSpec((D, PCOL), lambda i: (0, i + nblk))
    eye_spec = pl.BlockSpec((D, D), lambda i: (0, 0))
    out_spec = pl.BlockSpec((PCOL, 2 * D), lambda i: (i, 0))
    out_shape = jax.ShapeDtypeStruct((HALF, 2 * D), jnp.float32)
    return pl.pallas_call(
        _pack_body,
        grid=(nblk,),
        in_specs=[lo_spec, hi_spec, lo_spec, hi_spec, eye_spec],
        out_specs=[out_spec, out_spec],
        out_shape=[out_shape, out_shape],
    )(utT, utT, itT, itT, jnp.eye(D, dtype=jnp.float32))


def _sc_gather(uid, iid, ut2, it2):
    mesh = plsc.VectorSubcoreMesh(core_axis_name="c", subcore_axis_name="s")

    @functools.partial(
        pl.kernel,
        mesh=mesh,
        compiler_params=pltpu.CompilerParams(use_tc_tiling_on_sc=True),
        out_type=[
            jax.ShapeDtypeStruct((B, 2 * D), jnp.float32),
            jax.ShapeDtypeStruct((B, 2 * D), jnp.float32),
        ],
        scratch_types=[
            pltpu.VMEM((CB,), jnp.int32),
            pltpu.VMEM((CB,), jnp.int32),
            pltpu.VMEM((CB, 2 * D), jnp.float32),
            pltpu.VMEM((CB, 2 * D), jnp.float32),
            pltpu.SemaphoreType.DMA,
            pltpu.SemaphoreType.DMA,
            pltpu.SemaphoreType.DMA,
            pltpu.SemaphoreType.DMA,
        ],
    )
    def k(uid_hbm, iid_hbm, ut_hbm, it_hbm, uout_hbm, iout_hbm,
          uidx_v, iidx_v, urows_v, irows_v, gu, gi, wu, wi):
        wid = lax.axis_index("s") * NC + lax.axis_index("c")

        def fold(iv):
            def body(i, _):
                x = iv[pl.ds(i * 16, 16)]
                iv[pl.ds(i * 16, 16)] = jnp.where(x < HALF, x, x - HALF)
                return 0
            lax.fori_loop(0, CB // 16, body, 0)

        for c in range(2):
            base = wid * B_PER_W + c * CB
            pltpu.sync_copy(uid_hbm.at[pl.ds(base, CB)], uidx_v)
            fold(uidx_v)
            cu = pltpu.async_copy(ut_hbm.at[uidx_v], urows_v, gu)
            pltpu.sync_copy(iid_hbm.at[pl.ds(base, CB)], iidx_v)
            fold(iidx_v)
            ci = pltpu.async_copy(it_hbm.at[iidx_v], irows_v, gi)
            cu.wait()
            su = pltpu.async_copy(urows_v, uout_hbm.at[pl.ds(base, CB)], wu)
            ci.wait()
            si = pltpu.async_copy(irows_v, iout_hbm.at[pl.ds(base, CB)], wi)
            su.wait()
            si.wait()

    return k(uid, iid, ut2, it2)


def _mlp_body(urows, irows, uidc, iidc, uw1, ub1, uw2, ub2,
              iw1, ib1, iw2, ib2, out):
    ur = jnp.where(uidc[:] < HALF, urows[:, :D], urows[:, D:])
    ir = jnp.where(iidc[:] < HALF, irows[:, :D], irows[:, D:])
    u = jnp.dot(ur, uw1[:], preferred_element_type=jnp.float32) + ub1[:]
    u = jnp.maximum(u, 0.0)
    u = jnp.dot(u, uw2[:], preferred_element_type=jnp.float32) + ub2[:]
    u = jnp.maximum(u, 0.0)
    v = jnp.dot(ir, iw1[:], preferred_element_type=jnp.float32) + ib1[:]
    v = jnp.maximum(v, 0.0)
    v = jnp.dot(v, iw2[:], preferred_element_type=jnp.float32) + ib2[:]
    v = jnp.maximum(v, 0.0)
    out[:] = jnp.maximum(jnp.sum(u * v, axis=1), 0.0).reshape(out.shape)


BLK = 1024


def _tc_mlp(urows, irows, uid, iid, uW1, ub1, uW2, ub2, iW1, ib1, iW2, ib2):
    nblk = B // BLK
    row_spec = pl.BlockSpec((BLK, 2 * D), lambda i: (i, 0))
    idc_spec = pl.BlockSpec((BLK, 1), lambda i: (i, 0))
    w1_spec = pl.BlockSpec((D, H1), lambda i: (0, 0))
    b1_spec = pl.BlockSpec((1, H1), lambda i: (0, 0))
    w2_spec = pl.BlockSpec((H1, H2), lambda i: (0, 0))
    b2_spec = pl.BlockSpec((1, H2), lambda i: (0, 0))
    out = pl.pallas_call(
        _mlp_body,
        grid=(nblk,),
        in_specs=[row_spec, row_spec, idc_spec, idc_spec,
                  w1_spec, b1_spec, w2_spec, b2_spec,
                  w1_spec, b1_spec, w2_spec, b2_spec],
        out_specs=pl.BlockSpec((BLK // 128, 128), lambda i: (i, 0)),
        out_shape=jax.ShapeDtypeStruct((B // 128, 128), jnp.float32),
    )(urows, irows, uid.reshape(B, 1), iid.reshape(B, 1),
      uW1, ub1.reshape(1, H1), uW2, ub2.reshape(1, H2),
      iW1, ib1.reshape(1, H1), iW2, ib2.reshape(1, H2))
    return out.reshape(-1)


def kernel(uid, iid, user_table, uW1, ub1, uW2, ub2, item_table, iW1, ib1, iW2, ib2):
    uid = uid.astype(jnp.int32)
    iid = iid.astype(jnp.int32)
    ut2, it2 = _tc_pack(user_table.T, item_table.T)
    urows, irows = _sc_gather(uid, iid, ut2, it2)
    return _tc_mlp(urows, irows, uid, iid,
                   uW1, ub1, uW2, ub2, iW1, ib1, iW2, ib2)
